# baseline (speedup 1.0000x reference)
import functools

import jax
import jax.numpy as jnp
from jax.experimental import pallas as pl
from jax.experimental.pallas import tpu as pltpu


def _w6(w_pt):
    cin, cout = w_pt.shape[0], w_pt.shape[1]
    kidx = jnp.array([[2, 3], [0, 1]], dtype=jnp.int32)
    w = w_pt[:, :, kidx, :]
    w = w[:, :, :, :, kidx]
    w = jnp.transpose(w, (2, 4, 0, 3, 5, 1))
    return w.reshape(2, 2, cin, 4 * cout)


def _fused_kernel(p0_ref, w0_ref, b0_ref, m_ref, g_ref, be_ref, cg_ref,
                  gc_ref, w1_ref, b1_ref, o_ref, h0_ref, h1_ref,
                  *, R, PAD, shifts, eps, inv_count):
    for img, h_ref in ((0, h0_ref), (1, h1_ref)):
        y = b0_ref[...] + jnp.dot(p0_ref[img], w0_ref[...],
                                  preferred_element_type=jnp.float32)

        m = m_ref[...]
        ym = y * m
        s = jnp.sum(ym, axis=0, keepdims=True)
        q = jnp.sum(ym * y, axis=0, keepdims=True)
        mean = jnp.dot(jnp.dot(s, cg_ref[...],
                               preferred_element_type=jnp.float32)
                       * inv_count, gc_ref[...],
                       preferred_element_type=jnp.float32)
        msq = jnp.dot(jnp.dot(q, cg_ref[...],
                              preferred_element_type=jnp.float32)
                      * inv_count, gc_ref[...],
                      preferred_element_type=jnp.float32)
        inv = jax.lax.rsqrt(msq - mean * mean + eps)
        ga = inv * g_ref[...]
        gb = be_ref[...] - mean * ga
        yn = y * ga + gb
        act = 0.5 * yn * (
            1.0 + jax.lax.erf(yn * jnp.float32(0.7071067811865476)))
        hh = (act * m).astype(h_ref.dtype)
        C4 = hh.shape[1]
        hh_dn = jnp.concatenate(
            [hh[1:], jnp.zeros((1, C4), h_ref.dtype)], axis=0)
        h_ref[pl.ds(0, R), 0:C4] = hh
        h_ref[pl.ds(0, R), C4:2 * C4] = hh_dn
        h_ref[pl.ds(R, PAD), :] = jnp.zeros((PAD, 2 * C4), h_ref.dtype)

        acc = jnp.dot(h_ref[pl.ds(0, R), :], w1_ref[0],
                      preferred_element_type=jnp.float32)
        acc = acc + jnp.dot(h_ref[pl.ds(shifts[2], R), :], w1_ref[1],
                            preferred_element_type=jnp.float32)
        o_ref[img] = jax.nn.sigmoid(acc + b1_ref[...])


def kernel(z, fc_w, fc_b, deconv0_w, deconv0_b, deconv1_w, deconv1_b,
           gn0_g, gn0_b, groups=16, eps=1e-5):
    N, latent, H, _ = z.shape
    He = H + 2
    R = He * He
    c0 = deconv0_w.shape[1]
    c1 = deconv1_w.shape[1]
    C4 = 4 * c0
    NC = 16 * c1
    shifts = (0, 1, He, He + 1)
    Rs = ((R + He + 2 + 7) // 8) * 8
    PAD = Rs - R

    w6_0 = _w6(deconv0_w)
    w_fc = fc_w.reshape(fc_w.shape[0], latent).T
    w0 = jnp.einsum("le,abeD->ablD", w_fc, w6_0).reshape(4 * latent, C4)
    w0b = w0.astype(jnp.bfloat16)

    w6x = _w6(deconv1_w).reshape(2, 2, c0, 2, 2, c1)
    w1 = jnp.zeros((2, 2, 2, 2, c0, 2, 2, c1, 2, 2), jnp.float32)
    for rho in range(2):
        for sig in range(2):
            for a in range(2):
                for b in range(2):
                    dt, v = divmod(rho + a, 2)
                    ds, u = divmod(sig + b, 2)
                    blk = jnp.transpose(w6x[a, b], (0, 1, 3, 2))
                    w1 = w1.at[dt, ds, v, u, :, rho, :, :, sig, :].set(blk)
    w1 = w1.reshape(2, 2, C4, NC)
    w1 = jnp.concatenate([w1[:, 0], w1[:, 1]], axis=1)
    w1 = w1.astype(jnp.bfloat16)
    b1 = jnp.tile(jnp.repeat(deconv1_b, 4), 4)[None, :]

    idx = jnp.arange(He + 1)
    inter = ((idx >= 1) & (idx <= H)).astype(jnp.float32)
    b0 = jnp.zeros((He, He, C4), jnp.float32)
    for a in range(2):
        for b in range(2):
            m2 = jnp.outer(inter[a:a + He], inter[b:b + He])
            b0 = b0 + m2[:, :, None] * (fc_b @ w6_0[a, b])[None, None, :]
    b0 = (b0 + jnp.tile(deconv0_b, 4)[None, None, :]).reshape(R, C4)

    t = jnp.arange(He)
    vu = jnp.arange(2)
    ry = 2 * t[:, None] - 1 + vu[None, :]
    rv = (ry >= 0) & (ry < 2 * H)
    m4 = rv[:, None, :, None] & rv[None, :, None, :]
    mask = jnp.broadcast_to(m4[..., None].astype(jnp.float32),
                            (He, He, 2, 2, c0)).reshape(R, C4)

    gamma4 = jnp.tile(gn0_g, 4)[None, :]
    beta4 = jnp.tile(gn0_b, 4)[None, :]
    cpg = c0 // groups
    g_of = (jnp.arange(C4) % c0) // cpg
    oh_cg = (g_of[:, None] == jnp.arange(groups)[None, :]).astype(jnp.float32)
    inv_count = 1.0 / (4.0 * H * H * cpg)

    x = jnp.transpose(z.astype(jnp.bfloat16), (0, 2, 3, 1))
    zp = jnp.pad(x, ((0, 0), (1, 3), (1, 1), (0, 0)))
    zp = zp.reshape(N, (H + 4) * He, latent)
    p0 = jnp.concatenate([zp[:, d:d + R] for d in shifts], axis=-1)

    kern = functools.partial(_fused_kernel, R=R, PAD=PAD, shifts=shifts,
                             eps=eps, inv_count=inv_count)
    o = pl.pallas_call(
        kern,
        out_shape=jax.ShapeDtypeStruct((N, R, NC), jnp.float32),
        grid=(N // 2,),
        in_specs=[
            pl.BlockSpec((2, R, 4 * latent), lambda i: (i, 0, 0)),
            pl.BlockSpec((4 * latent, C4), lambda i: (0, 0)),
            pl.BlockSpec((R, C4), lambda i: (0, 0)),
            pl.BlockSpec((R, C4), lambda i: (0, 0)),
            pl.BlockSpec((1, C4), lambda i: (0, 0)),
            pl.BlockSpec((1, C4), lambda i: (0, 0)),
            pl.BlockSpec((C4, groups), lambda i: (0, 0)),
            pl.BlockSpec((groups, C4), lambda i: (0, 0)),
            pl.BlockSpec((2, 2 * C4, NC), lambda i: (0, 0, 0)),
            pl.BlockSpec((1, NC), lambda i: (0, 0)),
        ],
        out_specs=pl.BlockSpec((2, R, NC), lambda i: (i, 0, 0)),
        scratch_shapes=[pltpu.VMEM((Rs, 2 * C4), jnp.bfloat16),
                        pltpu.VMEM((Rs, 2 * C4), jnp.bfloat16)],
        compiler_params=pltpu.CompilerParams(
            dimension_semantics=("parallel",)),
    )(p0, w0b, b0, mask, gamma4, beta4, oh_cg, oh_cg.T, w1, b1)

    o = o.reshape(N, He, He, 2, 2, c1, 2, 2)
    o = jnp.transpose(o, (0, 5, 1, 3, 4, 2, 6, 7))
    o = o.reshape(N, c1, 4 * He, 4 * He)
    return o[:, :, 1:4 * H + 1, 1:4 * H + 1]


# speedup vs baseline: 1.0175x; 1.0175x over previous
import functools

import jax
import jax.numpy as jnp
from jax.experimental import pallas as pl
from jax.experimental.pallas import tpu as pltpu


def _w6(w_pt):
    cin, cout = w_pt.shape[0], w_pt.shape[1]
    kidx = jnp.array([[2, 3], [0, 1]], dtype=jnp.int32)
    w = w_pt[:, :, kidx, :]
    w = w[:, :, :, :, kidx]
    w = jnp.transpose(w, (2, 4, 0, 3, 5, 1))
    return w.reshape(2, 2, cin, 4 * cout)


def _fused_kernel(p0_ref, w0_ref, b0_ref, m_ref, g_ref, be_ref, cg_ref,
                  gc_ref, w1_ref, b1_ref, o_ref, h_ref,
                  *, R, PAD, shifts, eps, inv_count):
    y = b0_ref[...] + jnp.dot(p0_ref[0], w0_ref[...],
                              preferred_element_type=jnp.float32)

    m = m_ref[...]
    ym = y * m
    s = jnp.sum(ym, axis=0, keepdims=True)
    q = jnp.sum(ym * y, axis=0, keepdims=True)
    mean = jnp.dot(jnp.dot(s, cg_ref[...], preferred_element_type=jnp.float32)
                   * inv_count, gc_ref[...],
                   preferred_element_type=jnp.float32)
    msq = jnp.dot(jnp.dot(q, cg_ref[...], preferred_element_type=jnp.float32)
                  * inv_count, gc_ref[...],
                  preferred_element_type=jnp.float32)
    inv = jax.lax.rsqrt(msq - mean * mean + eps)
    ga = inv * g_ref[...]
    gb = be_ref[...] - mean * ga
    yn = y * ga + gb
    act = 0.5 * yn * (1.0 + jax.lax.erf(yn * jnp.float32(0.7071067811865476)))
    hh = (act * m).astype(h_ref.dtype)
    C4 = hh.shape[1]
    hh_dn = jnp.concatenate(
        [hh[1:], jnp.zeros((1, C4), h_ref.dtype)], axis=0)
    h_ref[pl.ds(0, R), 0:C4] = hh
    h_ref[pl.ds(0, R), C4:2 * C4] = hh_dn
    h_ref[pl.ds(R, PAD), :] = jnp.zeros((PAD, 2 * C4), h_ref.dtype)

    acc = jnp.dot(h_ref[pl.ds(0, R), :], w1_ref[0],
                  preferred_element_type=jnp.float32)
    acc = acc + jnp.dot(h_ref[pl.ds(shifts[2], R), :], w1_ref[1],
                        preferred_element_type=jnp.float32)
    o_ref[0] = jax.nn.sigmoid(acc + b1_ref[...])


def kernel(z, fc_w, fc_b, deconv0_w, deconv0_b, deconv1_w, deconv1_b,
           gn0_g, gn0_b, groups=16, eps=1e-5):
    N, latent, H, _ = z.shape
    He = H + 2
    R = He * He
    c0 = deconv0_w.shape[1]
    c1 = deconv1_w.shape[1]
    C4 = 4 * c0
    NC = 16 * c1
    shifts = (0, 1, He, He + 1)
    Rs = ((R + He + 2 + 7) // 8) * 8
    PAD = Rs - R

    w6_0 = _w6(deconv0_w)
    w_fc = fc_w.reshape(fc_w.shape[0], latent).T
    w0 = jnp.einsum("le,abeD->ablD", w_fc, w6_0).reshape(4 * latent, C4)
    w0b = w0.astype(jnp.bfloat16)

    w6x = _w6(deconv1_w).reshape(2, 2, c0, 2, 2, c1)
    w1 = jnp.zeros((2, 2, 2, 2, c0, 2, 2, c1, 2, 2), jnp.float32)
    for rho in range(2):
        for sig in range(2):
            for a in range(2):
                for b in range(2):
                    dt, v = divmod(rho + a, 2)
                    ds, u = divmod(sig + b, 2)
                    blk = jnp.transpose(w6x[a, b], (0, 1, 3, 2))
                    w1 = w1.at[dt, ds, v, u, :, rho, :, :, sig, :].set(blk)
    w1 = w1.reshape(2, 2, C4, NC)
    w1 = jnp.concatenate([w1[:, 0], w1[:, 1]], axis=1)
    w1 = w1.astype(jnp.bfloat16)
    b1 = jnp.tile(jnp.repeat(deconv1_b, 4), 4)[None, :]

    idx = jnp.arange(He + 1)
    inter = ((idx >= 1) & (idx <= H)).astype(jnp.float32)
    b0 = jnp.zeros((He, He, C4), jnp.float32)
    for a in range(2):
        for b in range(2):
            m2 = jnp.outer(inter[a:a + He], inter[b:b + He])
            b0 = b0 + m2[:, :, None] * (fc_b @ w6_0[a, b])[None, None, :]
    b0 = (b0 + jnp.tile(deconv0_b, 4)[None, None, :]).reshape(R, C4)

    t = jnp.arange(He)
    vu = jnp.arange(2)
    ry = 2 * t[:, None] - 1 + vu[None, :]
    rv = (ry >= 0) & (ry < 2 * H)
    m4 = rv[:, None, :, None] & rv[None, :, None, :]
    mask = jnp.broadcast_to(m4[..., None].astype(jnp.float32),
                            (He, He, 2, 2, c0)).reshape(R, C4)

    gamma4 = jnp.tile(gn0_g, 4)[None, :]
    beta4 = jnp.tile(gn0_b, 4)[None, :]
    cpg = c0 // groups
    g_of = (jnp.arange(C4) % c0) // cpg
    oh_cg = (g_of[:, None] == jnp.arange(groups)[None, :]).astype(jnp.float32)
    inv_count = 1.0 / (4.0 * H * H * cpg)

    x = jnp.transpose(z, (0, 2, 3, 1)).astype(jnp.bfloat16)
    zp = jnp.pad(x, ((0, 0), (1, 3), (1, 1), (0, 0)))
    zp = zp.reshape(N, (H + 4) * He, latent)
    p0 = jnp.concatenate([zp[:, d:d + R] for d in shifts], axis=-1)

    kern = functools.partial(_fused_kernel, R=R, PAD=PAD, shifts=shifts,
                             eps=eps, inv_count=inv_count)
    o = pl.pallas_call(
        kern,
        out_shape=jax.ShapeDtypeStruct((N, R, NC), jnp.float32),
        grid=(N,),
        in_specs=[
            pl.BlockSpec((1, R, 4 * latent), lambda i: (i, 0, 0)),
            pl.BlockSpec((4 * latent, C4), lambda i: (0, 0)),
            pl.BlockSpec((R, C4), lambda i: (0, 0)),
            pl.BlockSpec((R, C4), lambda i: (0, 0)),
            pl.BlockSpec((1, C4), lambda i: (0, 0)),
            pl.BlockSpec((1, C4), lambda i: (0, 0)),
            pl.BlockSpec((C4, groups), lambda i: (0, 0)),
            pl.BlockSpec((groups, C4), lambda i: (0, 0)),
            pl.BlockSpec((2, 2 * C4, NC), lambda i: (0, 0, 0)),
            pl.BlockSpec((1, NC), lambda i: (0, 0)),
        ],
        out_specs=pl.BlockSpec((1, R, NC), lambda i: (i, 0, 0)),
        scratch_shapes=[pltpu.VMEM((Rs, 2 * C4), jnp.bfloat16)],
        compiler_params=pltpu.CompilerParams(
            dimension_semantics=("parallel",)),
    )(p0, w0b, b0, mask, gamma4, beta4, oh_cg, oh_cg.T, w1, b1)

    o = o.reshape(N, He, He, 2, 2, c1, 2, 2)
    o = jnp.transpose(o, (0, 5, 1, 3, 4, 2, 6, 7))
    o = o.reshape(N, c1, 4 * He, 4 * He)
    return o[:, :, 1:4 * H + 1, 1:4 * H + 1]


# speedup vs baseline: 1.0921x; 1.0733x over previous
import functools

import jax
import jax.numpy as jnp
from jax.experimental import pallas as pl
from jax.experimental.pallas import tpu as pltpu


def _w6(w_pt):
    cin, cout = w_pt.shape[0], w_pt.shape[1]
    kidx = jnp.array([[2, 3], [0, 1]], dtype=jnp.int32)
    w = w_pt[:, :, kidx, :]
    w = w[:, :, :, :, kidx]
    w = jnp.transpose(w, (2, 4, 0, 3, 5, 1))
    return w.reshape(2, 2, cin, 4 * cout)


def _fused_kernel(p0_ref, w0_ref, b0_ref, m_ref, g_ref, be_ref, cg_ref,
                  gc_ref, w1_ref, b1_ref, o_ref, h_ref,
                  *, R, PAD, shifts, eps, inv_count):
    y = b0_ref[...] + jnp.dot(p0_ref[0], w0_ref[...],
                              preferred_element_type=jnp.float32)

    m = m_ref[...]
    ym = y * m
    s = jnp.sum(ym, axis=0, keepdims=True)
    q = jnp.sum(ym * y, axis=0, keepdims=True)
    mean = jnp.dot(jnp.dot(s, cg_ref[...], preferred_element_type=jnp.float32)
                   * inv_count, gc_ref[...],
                   preferred_element_type=jnp.float32)
    msq = jnp.dot(jnp.dot(q, cg_ref[...], preferred_element_type=jnp.float32)
                  * inv_count, gc_ref[...],
                  preferred_element_type=jnp.float32)
    inv = jax.lax.rsqrt(msq - mean * mean + eps)
    ga = inv * g_ref[...]
    gb = be_ref[...] - mean * ga
    yn = y * ga + gb
    act = 0.5 * yn * (1.0 + jax.lax.erf(yn * jnp.float32(0.7071067811865476)))
    hh = (act * m).astype(h_ref.dtype)
    C4 = hh.shape[1]
    hh_dn = jnp.concatenate(
        [hh[1:], jnp.zeros((1, C4), h_ref.dtype)], axis=0)
    h_ref[pl.ds(0, R), 0:C4] = hh
    h_ref[pl.ds(0, R), C4:2 * C4] = hh_dn
    h_ref[pl.ds(R, PAD), :] = jnp.zeros((PAD, 2 * C4), h_ref.dtype)

    acc = jnp.dot(h_ref[pl.ds(0, R), :], w1_ref[0],
                  preferred_element_type=jnp.float32)
    acc = acc + jnp.dot(h_ref[pl.ds(shifts[2], R), :], w1_ref[1],
                        preferred_element_type=jnp.float32)
    o_ref[0] = jax.nn.sigmoid(acc + b1_ref[...])


def kernel(z, fc_w, fc_b, deconv0_w, deconv0_b, deconv1_w, deconv1_b,
           gn0_g, gn0_b, groups=16, eps=1e-5):
    N, latent, H, _ = z.shape
    He = H + 2
    R = He * He
    c0 = deconv0_w.shape[1]
    c1 = deconv1_w.shape[1]
    C4 = 4 * c0
    NC = 16 * c1
    shifts = (0, 1, He, He + 1)
    Rp = ((R + 7) // 8) * 8
    Rs = ((Rp + He + 2 + 7) // 8) * 8
    PAD = Rs - Rp

    w6_0 = _w6(deconv0_w)
    w_fc = fc_w.reshape(fc_w.shape[0], latent).T
    w0 = jnp.einsum("le,abeD->ablD", w_fc, w6_0).reshape(4 * latent, C4)
    w0b = w0.astype(jnp.bfloat16)

    w6x = _w6(deconv1_w).reshape(2, 2, c0, 2, 2, c1)
    w1 = jnp.zeros((2, 2, 2, 2, c0, 2, 2, c1, 2, 2), jnp.float32)
    for rho in range(2):
        for sig in range(2):
            for a in range(2):
                for b in range(2):
                    dt, v = divmod(rho + a, 2)
                    ds, u = divmod(sig + b, 2)
                    blk = jnp.transpose(w6x[a, b], (0, 1, 3, 2))
                    w1 = w1.at[dt, ds, v, u, :, rho, :, :, sig, :].set(blk)
    w1 = w1.reshape(2, 2, C4, NC)
    w1 = jnp.concatenate([w1[:, 0], w1[:, 1]], axis=1)
    w1 = w1.astype(jnp.bfloat16)
    b1 = jnp.tile(jnp.repeat(deconv1_b, 4), 4)[None, :]

    idx = jnp.arange(He + 1)
    inter = ((idx >= 1) & (idx <= H)).astype(jnp.float32)
    b0 = jnp.zeros((He, He, C4), jnp.float32)
    for a in range(2):
        for b in range(2):
            m2 = jnp.outer(inter[a:a + He], inter[b:b + He])
            b0 = b0 + m2[:, :, None] * (fc_b @ w6_0[a, b])[None, None, :]
    b0 = (b0 + jnp.tile(deconv0_b, 4)[None, None, :]).reshape(R, C4)
    b0 = jnp.pad(b0, ((0, Rp - R), (0, 0)))

    t = jnp.arange(He)
    vu = jnp.arange(2)
    ry = 2 * t[:, None] - 1 + vu[None, :]
    rv = (ry >= 0) & (ry < 2 * H)
    m4 = rv[:, None, :, None] & rv[None, :, None, :]
    mask = jnp.broadcast_to(m4[..., None].astype(jnp.float32),
                            (He, He, 2, 2, c0)).reshape(R, C4)
    mask = jnp.pad(mask, ((0, Rp - R), (0, 0)))

    gamma4 = jnp.tile(gn0_g, 4)[None, :]
    beta4 = jnp.tile(gn0_b, 4)[None, :]
    cpg = c0 // groups
    g_of = (jnp.arange(C4) % c0) // cpg
    oh_cg = (g_of[:, None] == jnp.arange(groups)[None, :]).astype(jnp.float32)
    inv_count = 1.0 / (4.0 * H * H * cpg)

    x = jnp.transpose(z, (0, 2, 3, 1)).astype(jnp.bfloat16)
    zp = jnp.pad(x, ((0, 0), (1, 3), (1, 1), (0, 0)))
    zp = zp.reshape(N, (H + 4) * He, latent)
    p0 = jnp.concatenate([zp[:, d:d + Rp] for d in shifts], axis=-1)

    kern = functools.partial(_fused_kernel, R=Rp, PAD=PAD, shifts=shifts,
                             eps=eps, inv_count=inv_count)
    o = pl.pallas_call(
        kern,
        out_shape=jax.ShapeDtypeStruct((N, Rp, NC), jnp.float32),
        grid=(N,),
        in_specs=[
            pl.BlockSpec((1, Rp, 4 * latent), lambda i: (i, 0, 0)),
            pl.BlockSpec((4 * latent, C4), lambda i: (0, 0)),
            pl.BlockSpec((Rp, C4), lambda i: (0, 0)),
            pl.BlockSpec((Rp, C4), lambda i: (0, 0)),
            pl.BlockSpec((1, C4), lambda i: (0, 0)),
            pl.BlockSpec((1, C4), lambda i: (0, 0)),
            pl.BlockSpec((C4, groups), lambda i: (0, 0)),
            pl.BlockSpec((groups, C4), lambda i: (0, 0)),
            pl.BlockSpec((2, 2 * C4, NC), lambda i: (0, 0, 0)),
            pl.BlockSpec((1, NC), lambda i: (0, 0)),
        ],
        out_specs=pl.BlockSpec((1, Rp, NC), lambda i: (i, 0, 0)),
        scratch_shapes=[pltpu.VMEM((Rs, 2 * C4), jnp.bfloat16)],
        compiler_params=pltpu.CompilerParams(
            dimension_semantics=("parallel",)),
    )(p0, w0b, b0, mask, gamma4, beta4, oh_cg, oh_cg.T, w1, b1)

    o = o[:, :R].reshape(N, He, He, 2, 2, c1, 2, 2)
    o = jnp.transpose(o, (0, 5, 1, 3, 4, 2, 6, 7))
    o = o.reshape(N, c1, 4 * He, 4 * He)
    return o[:, :, 1:4 * H + 1, 1:4 * H + 1]
